# baseline (speedup 1.0000x reference)
import functools
import math

import jax
import jax.numpy as jnp
from jax import lax
from jax.experimental import pallas as pl
from jax.experimental.pallas import tpu as pltpu
from jax.experimental.pallas import tpu_sc as plsc

_N = 10000
_E = 320000
_D = 128
_T = 32

_NC = 2
_NS = 16
_NW = _NC * _NS
_EW = _E // _NW
_CK = 80
_CH = -(-_EW // _CK)
_EWP = _CH * _CK
_NP = 10240
_RPT = _NP // _NS


def _sc_agg_body(h_hbm, pk_hbm, out_hbm,
                 pk_v, srci_v, dsti_v, rows_v, agg_s, sem, ssem):
    cid = lax.axis_index("c")
    sid = lax.axis_index("s")
    w = sid * _NC + cid

    pltpu.sync_copy(pk_hbm.at[w], pk_v)

    def _unpack(j, slot):
        for g in range(_CK // 16):
            p = pk_v[j, pl.ds(g * 16, 16)]
            srci_v[slot, pl.ds(g * 16, 16)] = lax.bitwise_and(p, 0xFFFF)
            dsti_v[slot, pl.ds(g * 16, 16)] = lax.shift_right_logical(p, 16)

    _unpack(0, 0)
    pltpu.async_copy(h_hbm.at[srci_v.at[0]], rows_v.at[0], sem.at[0])

    def _zrow(r, carry):
        for c8 in range(_D // 16):
            rows_v[1, r, pl.ds(c8 * 16, 16)] = jnp.zeros((16,), jnp.float32)
        return carry

    lax.fori_loop(0, _CK, _zrow, 0)
    for k in range(_RPT // _CK):
        pltpu.sync_copy(rows_v.at[1], agg_s.at[pl.ds(sid * _RPT + k * _CK, _CK)])
    plsc.subcore_barrier()

    def _step2(i, carry):
        for b in range(2):
            j = i * 2 + b
            pltpu.make_async_copy(h_hbm.at[srci_v.at[b]], rows_v.at[b],
                                  sem.at[b]).wait()
            pltpu.async_copy(rows_v.at[b], agg_s.at[dsti_v.at[b]],
                             ssem.at[b], add=True)

            @pl.when(j >= 1)
            def _():
                pltpu.make_async_copy(rows_v.at[1 - b],
                                      agg_s.at[dsti_v.at[1 - b]],
                                      ssem.at[1 - b]).wait()

            @pl.when(j < _CH - 1)
            def _():
                _unpack(j + 1, 1 - b)
                pltpu.async_copy(h_hbm.at[srci_v.at[1 - b]], rows_v.at[1 - b],
                                 sem.at[1 - b])
        return carry

    lax.fori_loop(0, _CH // 2, _step2, 0)
    if _CH % 2:
        pltpu.make_async_copy(h_hbm.at[srci_v.at[0]], rows_v.at[0],
                              sem.at[0]).wait()
        pltpu.async_copy(rows_v.at[0], agg_s.at[dsti_v.at[0]],
                         ssem.at[0], add=True)
        pltpu.make_async_copy(rows_v.at[1], agg_s.at[dsti_v.at[1]],
                              ssem.at[1]).wait()
        pltpu.make_async_copy(rows_v.at[0], agg_s.at[dsti_v.at[0]],
                              ssem.at[0]).wait()
    else:
        pltpu.make_async_copy(rows_v.at[1], agg_s.at[dsti_v.at[1]],
                              ssem.at[1]).wait()
    plsc.subcore_barrier()

    pltpu.sync_copy(agg_s.at[pl.ds(sid * _RPT, _RPT)],
                    out_hbm.at[cid, pl.ds(sid * _RPT, _RPT)])


_sc_agg = pl.kernel(
    _sc_agg_body,
    out_type=jax.ShapeDtypeStruct((_NC, _NP, _D), jnp.float32),
    mesh=plsc.VectorSubcoreMesh(core_axis_name="c", subcore_axis_name="s",
                                num_cores=_NC, num_subcores=_NS),
    scratch_types=[
        pltpu.VMEM((_CH, _CK), jnp.int32),
        pltpu.VMEM((2, _CK), jnp.int32),
        pltpu.VMEM((2, _CK), jnp.int32),
        pltpu.VMEM((2, _CK, _D), jnp.float32),
        pltpu.VMEM_SHARED((_NP, _D), jnp.float32),
        pltpu.SemaphoreType.DMA((2,)),
        pltpu.SemaphoreType.DMA((2,)),
    ],
)


_BN = 1000


def _conv_body(epi, h_ref, p_ref, wr_ref, wn_ref, b_ref, g_ref, bb_ref,
               tv_ref, o_ref):
    acc = jnp.dot(h_ref[...], wr_ref[...],
                  preferred_element_type=jnp.float32,
                  precision=lax.Precision.DEFAULT)
    agg = p_ref[0] + p_ref[1]
    acc = acc + jnp.dot(agg, wn_ref[...],
                        preferred_element_type=jnp.float32,
                        precision=lax.Precision.DEFAULT)
    acc = acc + b_ref[...]
    if epi >= 1:
        acc = jnp.maximum(acc, 0.0)
    if epi >= 2:
        m = jnp.mean(acc, axis=-1, keepdims=True)
        v = jnp.mean((acc - m) ** 2, axis=-1, keepdims=True)
        acc = (acc - m) * lax.rsqrt(v + 1e-5) * g_ref[...] + bb_ref[...]
    if epi == 2:
        acc = acc + tv_ref[...]
    o_ref[...] = acc


def _make_conv(epi):
    return pl.pallas_call(
        functools.partial(_conv_body, epi),
        grid=(_N // _BN,),
        in_specs=[
            pl.BlockSpec((_BN, _D), lambda i: (i, 0)),
            pl.BlockSpec((_NC, _BN, _D), lambda i: (0, i, 0)),
            pl.BlockSpec((_D, _D), lambda i: (0, 0)),
            pl.BlockSpec((_D, _D), lambda i: (0, 0)),
            pl.BlockSpec((1, _D), lambda i: (0, 0)),
            pl.BlockSpec((1, _D), lambda i: (0, 0)),
            pl.BlockSpec((1, _D), lambda i: (0, 0)),
            pl.BlockSpec((1, _D), lambda i: (0, 0)),
        ],
        out_specs=pl.BlockSpec((_BN, _D), lambda i: (i, 0)),
        out_shape=jax.ShapeDtypeStruct((_N, _D), jnp.float32),
    )


_conv_plain = _make_conv(0)
_conv_relu = _make_conv(1)
_conv_relu_ln_tv = _make_conv(2)
_conv_relu_ln = _make_conv(3)


def _time_body(ts_ref, w_ref, b_ref, tw_ref, tb_ref, o_ref):
    t = ts_ref[0].astype(jnp.float32)
    half = _T // 2
    scale = math.log(10000.0) / (half - 1)
    idx = lax.broadcasted_iota(jnp.int32, (1, half), 1).astype(jnp.float32)
    freqs = jnp.exp(idx * -scale)
    args = t * freqs
    emb = jnp.concatenate([jnp.sin(args), jnp.cos(args)], axis=-1)
    mlp = jnp.maximum(
        jnp.dot(emb, w_ref[...], preferred_element_type=jnp.float32,
                precision=lax.Precision.DEFAULT) + b_ref[...], 0.0)
    rows = [
        jnp.dot(mlp, tw_ref[i], preferred_element_type=jnp.float32,
                precision=lax.Precision.DEFAULT) + tb_ref[i][None, :]
        for i in range(2)
    ]
    o_ref[...] = jnp.concatenate(rows, axis=0)


_time_vecs = pl.pallas_call(
    _time_body,
    in_specs=[
        pl.BlockSpec(memory_space=pltpu.SMEM),
        pl.BlockSpec((_T, _T), lambda: (0, 0)),
        pl.BlockSpec((1, _T), lambda: (0, 0)),
        pl.BlockSpec((2, _T, _D), lambda: (0, 0, 0)),
        pl.BlockSpec((2, _D), lambda: (0, 0)),
    ],
    out_specs=pl.BlockSpec((2, _D), lambda: (0, 0)),
    out_shape=jax.ShapeDtypeStruct((2, _D), jnp.float32),
)


def kernel(x, edge_index, timestep, conv_Wr, conv_Wn, conv_b, conv2_Wr,
           conv2_Wn, conv2_b, time_lin_W, time_lin_b, times_W, times_b,
           ln1_g, ln1_b, ln2_g, ln2_b):
    pk = (edge_index[0] | (edge_index[1] << 16)).reshape(_NW, _EW)
    pad = jnp.full((_NW, _EWP - _EW), _N << 16, jnp.int32)
    packed = jnp.concatenate([pk, pad], axis=1).reshape(_NW, _CH, _CK)

    tv = _time_vecs(timestep, time_lin_W, time_lin_b.reshape(1, _T),
                    times_W, times_b)

    ones = jnp.ones((1, _D), jnp.float32)
    zeros = jnp.zeros((1, _D), jnp.float32)

    def conv(h, Wr, Wn, b, epi, g=None, bb=None, tvrow=None):
        parts = _sc_agg(h, packed)
        fn = (_conv_plain, _conv_relu, _conv_relu_ln_tv, _conv_relu_ln)[epi]
        return fn(h, parts, Wr, Wn, b.reshape(1, _D),
                  ones if g is None else g.reshape(1, _D),
                  zeros if bb is None else bb.reshape(1, _D),
                  zeros if tvrow is None else tvrow.reshape(1, _D))

    h = x
    h = conv(h, conv_Wr[0], conv_Wn[0], conv_b[0], 1)
    h = conv(h, conv_Wr[1], conv_Wn[1], conv_b[1], 2,
             ln1_g[0], ln1_b[0], tv[0])
    h = conv(h, conv2_Wr[0], conv2_Wn[0], conv2_b[0], 3,
             ln2_g[0], ln2_b[0])
    h = conv(h, conv_Wr[2], conv_Wn[2], conv_b[2], 2,
             ln1_g[1], ln1_b[1], tv[1])
    h = conv(h, conv2_Wr[1], conv2_Wn[1], conv2_b[1], 3,
             ln2_g[1], ln2_b[1])
    h = conv(h, conv_Wr[3], conv_Wn[3], conv_b[3], 0)
    return h


# speedup vs baseline: 1.5731x; 1.5731x over previous
import functools
import math

import jax
import jax.numpy as jnp
from jax import lax
from jax.experimental import pallas as pl
from jax.experimental.pallas import tpu as pltpu
from jax.experimental.pallas import tpu_sc as plsc

_N = 10000
_E = 320000
_D = 128
_T = 32

_NC = 2
_NS = 16
_NW = _NC * _NS
_EW = _E // _NW
_CK = 80
_CH = -(-_EW // _CK)
_EWP = _CH * _CK
_NB = 3
_NP = 10112
_RPT = _NP // _NS
_ZB = _RPT // 8


def _sc_agg_body(h_hbm, pk_hbm, out_hbm,
                 pk_v, srci_v, dsti_v, rows_v, agg_s, sem):
    cid = lax.axis_index("c")
    sid = lax.axis_index("s")
    w = sid * _NC + cid

    pltpu.sync_copy(pk_hbm.at[w], pk_v)

    def _unpack(j, slot):
        for g in range(_CK // 16):
            p = pk_v[j, pl.ds(g * 16, 16)]
            srci_v[slot, pl.ds(g * 16, 16)] = lax.bitwise_and(p, 0xFFFF)
            dsti_v[slot, pl.ds(g * 16, 16)] = lax.shift_right_logical(p, 16)

    _unpack(0, 0)
    pltpu.async_copy(h_hbm.at[srci_v.at[0]], rows_v.at[0], sem.at[0])
    _unpack(1, 1)
    pltpu.async_copy(h_hbm.at[srci_v.at[1]], rows_v.at[1], sem.at[1])

    def _zrow(r, carry):
        for c8 in range(_D // 16):
            rows_v[_NB - 1, r, pl.ds(c8 * 16, 16)] = jnp.zeros(
                (16,), jnp.float32)
        return carry

    lax.fori_loop(0, _ZB, _zrow, 0)
    for k in range(_RPT // _ZB):
        pltpu.sync_copy(rows_v.at[_NB - 1, pl.ds(0, _ZB)],
                        agg_s.at[pl.ds(sid * _RPT + k * _ZB, _ZB)])
    plsc.subcore_barrier()

    def _stepn(i, carry):
        for b in range(_NB):
            j = i * _NB + b

            @pl.when(j < _CH - 2)
            def _():
                nb = (b + 2) % _NB
                _unpack(j + 2, nb)
                pltpu.async_copy(h_hbm.at[srci_v.at[nb]], rows_v.at[nb],
                                 sem.at[nb])

            pltpu.make_async_copy(h_hbm.at[srci_v.at[b]], rows_v.at[b],
                                  sem.at[b]).wait()
            pltpu.sync_copy(rows_v.at[b], agg_s.at[dsti_v.at[b]], add=True)
        return carry

    lax.fori_loop(0, _CH // _NB, _stepn, 0)
    for j in range(_CH - _CH % _NB, _CH):
        b = j % _NB
        pltpu.make_async_copy(h_hbm.at[srci_v.at[b]], rows_v.at[b],
                              sem.at[b]).wait()
        pltpu.sync_copy(rows_v.at[b], agg_s.at[dsti_v.at[b]], add=True)
    plsc.subcore_barrier()

    pltpu.sync_copy(agg_s.at[pl.ds(sid * _RPT, _RPT)],
                    out_hbm.at[cid, pl.ds(sid * _RPT, _RPT)])


_sc_agg = pl.kernel(
    _sc_agg_body,
    out_type=jax.ShapeDtypeStruct((_NC, _NP, _D), jnp.float32),
    mesh=plsc.VectorSubcoreMesh(core_axis_name="c", subcore_axis_name="s",
                                num_cores=_NC, num_subcores=_NS),
    scratch_types=[
        pltpu.VMEM((_CH, _CK), jnp.int32),
        pltpu.VMEM((_NB, _CK), jnp.int32),
        pltpu.VMEM((_NB, _CK), jnp.int32),
        pltpu.VMEM((_NB, _CK, _D), jnp.float32),
        pltpu.VMEM_SHARED((_NP, _D), jnp.float32),
        pltpu.SemaphoreType.DMA((_NB,)),
    ],
)


_BN = 1000


def _conv_body(epi, h_ref, p_ref, wr_ref, wn_ref, b_ref, g_ref, bb_ref,
               tv_ref, o_ref):
    acc = jnp.dot(h_ref[...], wr_ref[...],
                  preferred_element_type=jnp.float32,
                  precision=lax.Precision.DEFAULT)
    agg = p_ref[0] + p_ref[1]
    acc = acc + jnp.dot(agg, wn_ref[...],
                        preferred_element_type=jnp.float32,
                        precision=lax.Precision.DEFAULT)
    acc = acc + b_ref[...]
    if epi >= 1:
        acc = jnp.maximum(acc, 0.0)
    if epi >= 2:
        m = jnp.mean(acc, axis=-1, keepdims=True)
        v = jnp.mean((acc - m) ** 2, axis=-1, keepdims=True)
        acc = (acc - m) * lax.rsqrt(v + 1e-5) * g_ref[...] + bb_ref[...]
    if epi == 2:
        acc = acc + tv_ref[...]
    o_ref[...] = acc


def _make_conv(epi):
    return pl.pallas_call(
        functools.partial(_conv_body, epi),
        grid=(_N // _BN,),
        in_specs=[
            pl.BlockSpec((_BN, _D), lambda i: (i, 0)),
            pl.BlockSpec((_NC, _BN, _D), lambda i: (0, i, 0)),
            pl.BlockSpec((_D, _D), lambda i: (0, 0)),
            pl.BlockSpec((_D, _D), lambda i: (0, 0)),
            pl.BlockSpec((1, _D), lambda i: (0, 0)),
            pl.BlockSpec((1, _D), lambda i: (0, 0)),
            pl.BlockSpec((1, _D), lambda i: (0, 0)),
            pl.BlockSpec((1, _D), lambda i: (0, 0)),
        ],
        out_specs=pl.BlockSpec((_BN, _D), lambda i: (i, 0)),
        out_shape=jax.ShapeDtypeStruct((_N, _D), jnp.float32),
    )


_conv_plain = _make_conv(0)
_conv_relu = _make_conv(1)
_conv_relu_ln_tv = _make_conv(2)
_conv_relu_ln = _make_conv(3)


def _time_body(ts_ref, w_ref, b_ref, tw_ref, tb_ref, o_ref):
    t = ts_ref[0].astype(jnp.float32)
    half = _T // 2
    scale = math.log(10000.0) / (half - 1)
    idx = lax.broadcasted_iota(jnp.int32, (1, half), 1).astype(jnp.float32)
    freqs = jnp.exp(idx * -scale)
    args = t * freqs
    emb = jnp.concatenate([jnp.sin(args), jnp.cos(args)], axis=-1)
    mlp = jnp.maximum(
        jnp.dot(emb, w_ref[...], preferred_element_type=jnp.float32,
                precision=lax.Precision.DEFAULT) + b_ref[...], 0.0)
    rows = [
        jnp.dot(mlp, tw_ref[i], preferred_element_type=jnp.float32,
                precision=lax.Precision.DEFAULT) + tb_ref[i][None, :]
        for i in range(2)
    ]
    o_ref[...] = jnp.concatenate(rows, axis=0)


_time_vecs = pl.pallas_call(
    _time_body,
    in_specs=[
        pl.BlockSpec(memory_space=pltpu.SMEM),
        pl.BlockSpec((_T, _T), lambda: (0, 0)),
        pl.BlockSpec((1, _T), lambda: (0, 0)),
        pl.BlockSpec((2, _T, _D), lambda: (0, 0, 0)),
        pl.BlockSpec((2, _D), lambda: (0, 0)),
    ],
    out_specs=pl.BlockSpec((2, _D), lambda: (0, 0)),
    out_shape=jax.ShapeDtypeStruct((2, _D), jnp.float32),
)


def kernel(x, edge_index, timestep, conv_Wr, conv_Wn, conv_b, conv2_Wr,
           conv2_Wn, conv2_b, time_lin_W, time_lin_b, times_W, times_b,
           ln1_g, ln1_b, ln2_g, ln2_b):
    pk = (edge_index[0] | (edge_index[1] << 16)).reshape(_NW, _EW)
    pad = jnp.full((_NW, _EWP - _EW), _N << 16, jnp.int32)
    packed = jnp.concatenate([pk, pad], axis=1).reshape(_NW, _CH, _CK)

    tv = _time_vecs(timestep, time_lin_W, time_lin_b.reshape(1, _T),
                    times_W, times_b)

    ones = jnp.ones((1, _D), jnp.float32)
    zeros = jnp.zeros((1, _D), jnp.float32)

    def conv(h, Wr, Wn, b, epi, g=None, bb=None, tvrow=None):
        parts = _sc_agg(h, packed)
        fn = (_conv_plain, _conv_relu, _conv_relu_ln_tv, _conv_relu_ln)[epi]
        return fn(h, parts, Wr, Wn, b.reshape(1, _D),
                  ones if g is None else g.reshape(1, _D),
                  zeros if bb is None else bb.reshape(1, _D),
                  zeros if tvrow is None else tvrow.reshape(1, _D))

    h = x
    h = conv(h, conv_Wr[0], conv_Wn[0], conv_b[0], 1)
    h = conv(h, conv_Wr[1], conv_Wn[1], conv_b[1], 2,
             ln1_g[0], ln1_b[0], tv[0])
    h = conv(h, conv2_Wr[0], conv2_Wn[0], conv2_b[0], 3,
             ln2_g[0], ln2_b[0])
    h = conv(h, conv_Wr[2], conv_Wn[2], conv_b[2], 2,
             ln1_g[1], ln1_b[1], tv[1])
    h = conv(h, conv2_Wr[1], conv2_Wn[1], conv2_b[1], 3,
             ln2_g[1], ln2_b[1])
    h = conv(h, conv_Wr[3], conv_Wn[3], conv_b[3], 0)
    return h


# speedup vs baseline: 1.5775x; 1.0028x over previous
import functools
import math

import jax
import jax.numpy as jnp
from jax import lax
from jax.experimental import pallas as pl
from jax.experimental.pallas import tpu as pltpu
from jax.experimental.pallas import tpu_sc as plsc

_N = 10000
_E = 320000
_D = 128
_T = 32

_NC = 2
_NS = 16
_NW = _NC * _NS
_EW = _E // _NW
_CK = 80
_CH = -(-_EW // _CK)
_EWP = _CH * _CK
_NB = 4
_NP = 10112
_RPT = _NP // _NS
_ZB = _RPT // 8


def _sc_agg_body(h_hbm, pk_hbm, out_hbm,
                 pkr_v, srci_v, dsti_v, rows_v, agg_s, isem, sem):
    cid = lax.axis_index("c")
    sid = lax.axis_index("s")
    w = sid * _NC + cid

    def _idx_start(j, slot):
        pltpu.async_copy(pk_hbm.at[w, j], pkr_v.at[slot], isem.at[slot])

    def _idx_wait(j, slot):
        pltpu.make_async_copy(pk_hbm.at[w, j], pkr_v.at[slot],
                              isem.at[slot]).wait()

    def _unpack(slot):
        for g in range(_CK // 16):
            p = pkr_v[slot, pl.ds(g * 16, 16)]
            srci_v[slot, pl.ds(g * 16, 16)] = lax.bitwise_and(p, 0xFFFF)
            dsti_v[slot, pl.ds(g * 16, 16)] = lax.shift_right_logical(p, 16)

    def _gather_start(slot):
        pltpu.async_copy(h_hbm.at[srci_v.at[slot]], rows_v.at[slot],
                         sem.at[slot])

    def _gather_wait(slot):
        pltpu.make_async_copy(h_hbm.at[srci_v.at[slot]], rows_v.at[slot],
                              sem.at[slot]).wait()

    for j0 in range(_NB):
        _idx_start(j0, j0)
    for j0 in range(2):
        _idx_wait(j0, j0)
        _unpack(j0)
        _gather_start(j0)

    def _zrow(r, carry):
        for c8 in range(_D // 16):
            rows_v[_NB - 1, r, pl.ds(c8 * 16, 16)] = jnp.zeros(
                (16,), jnp.float32)
        return carry

    lax.fori_loop(0, _ZB, _zrow, 0)
    for k in range(_RPT // _ZB):
        pltpu.sync_copy(rows_v.at[_NB - 1, pl.ds(0, _ZB)],
                        agg_s.at[pl.ds(sid * _RPT + k * _ZB, _ZB)])
    plsc.subcore_barrier()

    def _stepn(i, carry):
        for b in range(_NB):
            j = i * _NB + b

            @pl.when(j + _NB < _CH)
            def _():
                _idx_start(j + _NB, b)

            @pl.when(j + 2 < _CH)
            def _():
                nb = (b + 2) % _NB
                _idx_wait(j + 2, nb)
                _unpack(nb)
                _gather_start(nb)

            _gather_wait(b)
            pltpu.sync_copy(rows_v.at[b], agg_s.at[dsti_v.at[b]], add=True)
        return carry

    lax.fori_loop(0, _CH // _NB, _stepn, 0)
    for j in range(_CH - _CH % _NB, _CH):
        b = j % _NB
        _gather_wait(b)
        pltpu.sync_copy(rows_v.at[b], agg_s.at[dsti_v.at[b]], add=True)
    plsc.subcore_barrier()

    pltpu.sync_copy(agg_s.at[pl.ds(sid * _RPT, _RPT)],
                    out_hbm.at[cid, pl.ds(sid * _RPT, _RPT)])


_sc_agg = pl.kernel(
    _sc_agg_body,
    out_type=jax.ShapeDtypeStruct((_NC, _NP, _D), jnp.float32),
    mesh=plsc.VectorSubcoreMesh(core_axis_name="c", subcore_axis_name="s",
                                num_cores=_NC, num_subcores=_NS),
    scratch_types=[
        pltpu.VMEM((_NB, _CK), jnp.int32),
        pltpu.VMEM((_NB, _CK), jnp.int32),
        pltpu.VMEM((_NB, _CK), jnp.int32),
        pltpu.VMEM((_NB, _CK, _D), jnp.float32),
        pltpu.VMEM_SHARED((_NP, _D), jnp.float32),
        pltpu.SemaphoreType.DMA((_NB,)),
        pltpu.SemaphoreType.DMA((_NB,)),
    ],
)


_BN = 1000


def _conv_body(epi, h_ref, p_ref, wr_ref, wn_ref, b_ref, g_ref, bb_ref,
               tv_ref, o_ref):
    acc = jnp.dot(h_ref[...], wr_ref[...],
                  preferred_element_type=jnp.float32,
                  precision=lax.Precision.DEFAULT)
    agg = p_ref[0] + p_ref[1]
    acc = acc + jnp.dot(agg, wn_ref[...],
                        preferred_element_type=jnp.float32,
                        precision=lax.Precision.DEFAULT)
    acc = acc + b_ref[...]
    if epi >= 1:
        acc = jnp.maximum(acc, 0.0)
    if epi >= 2:
        m = jnp.mean(acc, axis=-1, keepdims=True)
        v = jnp.mean((acc - m) ** 2, axis=-1, keepdims=True)
        acc = (acc - m) * lax.rsqrt(v + 1e-5) * g_ref[...] + bb_ref[...]
    if epi == 2:
        acc = acc + tv_ref[...]
    o_ref[...] = acc


def _make_conv(epi):
    return pl.pallas_call(
        functools.partial(_conv_body, epi),
        grid=(_N // _BN,),
        in_specs=[
            pl.BlockSpec((_BN, _D), lambda i: (i, 0)),
            pl.BlockSpec((_NC, _BN, _D), lambda i: (0, i, 0)),
            pl.BlockSpec((_D, _D), lambda i: (0, 0)),
            pl.BlockSpec((_D, _D), lambda i: (0, 0)),
            pl.BlockSpec((1, _D), lambda i: (0, 0)),
            pl.BlockSpec((1, _D), lambda i: (0, 0)),
            pl.BlockSpec((1, _D), lambda i: (0, 0)),
            pl.BlockSpec((1, _D), lambda i: (0, 0)),
        ],
        out_specs=pl.BlockSpec((_BN, _D), lambda i: (i, 0)),
        out_shape=jax.ShapeDtypeStruct((_N, _D), jnp.float32),
    )


_conv_plain = _make_conv(0)
_conv_relu = _make_conv(1)
_conv_relu_ln_tv = _make_conv(2)
_conv_relu_ln = _make_conv(3)


def _time_body(ts_ref, w_ref, b_ref, tw_ref, tb_ref, o_ref):
    t = ts_ref[0].astype(jnp.float32)
    half = _T // 2
    scale = math.log(10000.0) / (half - 1)
    idx = lax.broadcasted_iota(jnp.int32, (1, half), 1).astype(jnp.float32)
    freqs = jnp.exp(idx * -scale)
    args = t * freqs
    emb = jnp.concatenate([jnp.sin(args), jnp.cos(args)], axis=-1)
    mlp = jnp.maximum(
        jnp.dot(emb, w_ref[...], preferred_element_type=jnp.float32,
                precision=lax.Precision.DEFAULT) + b_ref[...], 0.0)
    rows = [
        jnp.dot(mlp, tw_ref[i], preferred_element_type=jnp.float32,
                precision=lax.Precision.DEFAULT) + tb_ref[i][None, :]
        for i in range(2)
    ]
    o_ref[...] = jnp.concatenate(rows, axis=0)


_time_vecs = pl.pallas_call(
    _time_body,
    in_specs=[
        pl.BlockSpec(memory_space=pltpu.SMEM),
        pl.BlockSpec((_T, _T), lambda: (0, 0)),
        pl.BlockSpec((1, _T), lambda: (0, 0)),
        pl.BlockSpec((2, _T, _D), lambda: (0, 0, 0)),
        pl.BlockSpec((2, _D), lambda: (0, 0)),
    ],
    out_specs=pl.BlockSpec((2, _D), lambda: (0, 0)),
    out_shape=jax.ShapeDtypeStruct((2, _D), jnp.float32),
)


def kernel(x, edge_index, timestep, conv_Wr, conv_Wn, conv_b, conv2_Wr,
           conv2_Wn, conv2_b, time_lin_W, time_lin_b, times_W, times_b,
           ln1_g, ln1_b, ln2_g, ln2_b):
    pk = (edge_index[0] | (edge_index[1] << 16)).reshape(_NW, _EW)
    pad = jnp.full((_NW, _EWP - _EW), _N << 16, jnp.int32)
    packed = jnp.concatenate([pk, pad], axis=1).reshape(_NW, _CH, _CK)

    tv = _time_vecs(timestep, time_lin_W, time_lin_b.reshape(1, _T),
                    times_W, times_b)

    ones = jnp.ones((1, _D), jnp.float32)
    zeros = jnp.zeros((1, _D), jnp.float32)

    def conv(h, Wr, Wn, b, epi, g=None, bb=None, tvrow=None):
        parts = _sc_agg(h, packed)
        fn = (_conv_plain, _conv_relu, _conv_relu_ln_tv, _conv_relu_ln)[epi]
        return fn(h, parts, Wr, Wn, b.reshape(1, _D),
                  ones if g is None else g.reshape(1, _D),
                  zeros if bb is None else bb.reshape(1, _D),
                  zeros if tvrow is None else tvrow.reshape(1, _D))

    h = x
    h = conv(h, conv_Wr[0], conv_Wn[0], conv_b[0], 1)
    h = conv(h, conv_Wr[1], conv_Wn[1], conv_b[1], 2,
             ln1_g[0], ln1_b[0], tv[0])
    h = conv(h, conv2_Wr[0], conv2_Wn[0], conv2_b[0], 3,
             ln2_g[0], ln2_b[0])
    h = conv(h, conv_Wr[2], conv_Wn[2], conv_b[2], 2,
             ln1_g[1], ln1_b[1], tv[1])
    h = conv(h, conv2_Wr[1], conv2_Wn[1], conv2_b[1], 3,
             ln2_g[1], ln2_b[1])
    h = conv(h, conv_Wr[3], conv_Wn[3], conv_b[3], 0)
    return h
